# baseline (speedup 1.0000x reference)
import functools

import jax
import jax.numpy as jnp
from jax.experimental import pallas as pl
from jax.experimental.pallas import tpu as pltpu

_NUM_EMBEDS = 1024
_EMBED_DIM = 256
_BETA = 0.25
_BPG = 2


def _vq_one_batch(x, emb2_bf, e_sq):
    n = x.shape[1]
    x_sq = jnp.sum(x * x, axis=0, keepdims=True)
    cross2 = jax.lax.dot_general(
        emb2_bf, x.astype(jnp.bfloat16),
        (((1,), (0,)), ((), ())),
        preferred_element_type=jnp.float32)
    dist = (x_sq + e_sq) - cross2
    minval = jnp.min(dist, axis=0, keepdims=True)
    iota_e = jax.lax.broadcasted_iota(jnp.int32, (_NUM_EMBEDS, n), 0)
    ind = jnp.min(jnp.where(dist == minval, iota_e, _NUM_EMBEDS),
                  axis=0, keepdims=True)
    onehot_half = jnp.where(iota_e == ind, 0.5, 0.0).astype(
        jnp.bfloat16)
    q_t = jax.lax.dot_general(
        emb2_bf, onehot_half, (((0,), (0,)), ((), ())),
        preferred_element_type=jnp.float32)
    return q_t, jnp.sum(minval)


def _vq_body(x_ref, e_ref, e2_ref, q_ref, loss_ref):
    g = pl.program_id(0)
    emb = e_ref[...]
    emb2_bf = e2_ref[...]
    e_sq = jnp.sum(emb * emb, axis=1, keepdims=True)
    partial = jnp.zeros((), jnp.float32)
    for i in range(_BPG):
        q_t, psum = _vq_one_batch(x_ref[i], emb2_bf, e_sq)
        q_ref[i] = q_t
        partial = partial + psum

    @pl.when(g == 0)
    def _init():
        loss_ref[...] = jnp.zeros((1, 1), jnp.float32)

    loss_ref[...] += partial.reshape(1, 1)


@functools.partial(jax.jit, static_argnames=())
def kernel(x_latent, embed_weight):
    B, C, H, W = x_latent.shape
    N = H * W
    x3 = x_latent.reshape(B, C, N)
    emb2_bf = (embed_weight * 2).astype(jnp.bfloat16)
    q3, loss_sum = pl.pallas_call(
        _vq_body,
        grid=(B // _BPG,),
        in_specs=[
            pl.BlockSpec((_BPG, C, N), lambda g: (g, 0, 0)),
            pl.BlockSpec((_NUM_EMBEDS, _EMBED_DIM), lambda g: (0, 0)),
            pl.BlockSpec((_NUM_EMBEDS, _EMBED_DIM), lambda g: (0, 0)),
        ],
        out_specs=[
            pl.BlockSpec((_BPG, C, N), lambda g: (g, 0, 0)),
            pl.BlockSpec((1, 1), lambda g: (0, 0)),
        ],
        out_shape=[
            jax.ShapeDtypeStruct((B, C, N), jnp.float32),
            jax.ShapeDtypeStruct((1, 1), jnp.float32),
        ],
    )(x3, embed_weight, emb2_bf)
    vq_loss = (1.0 + _BETA) * loss_sum[0, 0] / (B * C * H * W)
    return q3.reshape(B, C, H, W), vq_loss


# speedup vs baseline: 1.0195x; 1.0195x over previous
import functools

import jax
import jax.numpy as jnp
from jax.experimental import pallas as pl
from jax.experimental.pallas import tpu as pltpu

_NUM_EMBEDS = 1024
_EMBED_DIM = 256
_BETA = 0.25
_BPG = 4


def _vq_one_batch(x, emb2_bf, e_sq):
    n = x.shape[1]
    x_sq = jnp.sum(x * x, axis=0, keepdims=True)
    cross2 = jax.lax.dot_general(
        emb2_bf, x.astype(jnp.bfloat16),
        (((1,), (0,)), ((), ())),
        preferred_element_type=jnp.float32)
    dist = (x_sq + e_sq) - cross2
    minval = jnp.min(dist, axis=0, keepdims=True)
    iota_e = jax.lax.broadcasted_iota(jnp.int32, (_NUM_EMBEDS, n), 0)
    ind = jnp.min(jnp.where(dist == minval, iota_e, _NUM_EMBEDS),
                  axis=0, keepdims=True)
    onehot_half = jnp.where(iota_e == ind, 0.5, 0.0).astype(
        jnp.bfloat16)
    q_t = jax.lax.dot_general(
        emb2_bf, onehot_half, (((0,), (0,)), ((), ())),
        preferred_element_type=jnp.float32)
    return q_t, jnp.sum(minval)


def _vq_body(x_ref, e_ref, e2_ref, q_ref, loss_ref):
    g = pl.program_id(0)
    emb = e_ref[...]
    emb2_bf = e2_ref[...]
    e_sq = jnp.sum(emb * emb, axis=1, keepdims=True)
    partial = jnp.zeros((), jnp.float32)
    for i in range(_BPG):
        q_t, psum = _vq_one_batch(x_ref[i], emb2_bf, e_sq)
        q_ref[i] = q_t
        partial = partial + psum

    @pl.when(g == 0)
    def _init():
        loss_ref[...] = jnp.zeros((1, 1), jnp.float32)

    loss_ref[...] += partial.reshape(1, 1)


@functools.partial(jax.jit, static_argnames=())
def kernel(x_latent, embed_weight):
    B, C, H, W = x_latent.shape
    N = H * W
    x3 = x_latent.reshape(B, C, N)
    emb2_bf = (embed_weight * 2).astype(jnp.bfloat16)
    q3, loss_sum = pl.pallas_call(
        _vq_body,
        grid=(B // _BPG,),
        in_specs=[
            pl.BlockSpec((_BPG, C, N), lambda g: (g, 0, 0)),
            pl.BlockSpec((_NUM_EMBEDS, _EMBED_DIM), lambda g: (0, 0)),
            pl.BlockSpec((_NUM_EMBEDS, _EMBED_DIM), lambda g: (0, 0)),
        ],
        out_specs=[
            pl.BlockSpec((_BPG, C, N), lambda g: (g, 0, 0)),
            pl.BlockSpec((1, 1), lambda g: (0, 0)),
        ],
        out_shape=[
            jax.ShapeDtypeStruct((B, C, N), jnp.float32),
            jax.ShapeDtypeStruct((1, 1), jnp.float32),
        ],
    )(x3, embed_weight, emb2_bf)
    vq_loss = (1.0 + _BETA) * loss_sum[0, 0] / (B * C * H * W)
    return q3.reshape(B, C, H, W), vq_loss


# speedup vs baseline: 1.0208x; 1.0013x over previous
import functools

import jax
import jax.numpy as jnp
from jax.experimental import pallas as pl
from jax.experimental.pallas import tpu as pltpu

_NUM_EMBEDS = 1024
_EMBED_DIM = 256
_BETA = 0.25
_BPG = 4


def _vq_one_batch(x, emb2_bf, e_sq):
    n = x.shape[1]
    x_sq = jnp.sum(x * x, axis=0, keepdims=True)
    cross2 = jax.lax.dot_general(
        emb2_bf, x.astype(jnp.bfloat16),
        (((1,), (0,)), ((), ())),
        preferred_element_type=jnp.float32)
    dist = (x_sq + e_sq) - cross2
    minval = jnp.min(dist, axis=0, keepdims=True)
    iota_e = jax.lax.broadcasted_iota(jnp.int32, (_NUM_EMBEDS, n), 0)
    ind = jnp.min(jnp.where(dist == minval, iota_e, _NUM_EMBEDS),
                  axis=0, keepdims=True)
    iota16 = jax.lax.broadcasted_iota(jnp.int16, (_NUM_EMBEDS, n), 0)
    onehot_half = jnp.where(iota16 == ind.astype(jnp.int16),
                            jnp.bfloat16(0.5),
                            jnp.bfloat16(0.0))
    q_t = jax.lax.dot_general(
        emb2_bf, onehot_half, (((0,), (0,)), ((), ())),
        preferred_element_type=jnp.float32)
    return q_t, jnp.sum(minval)


def _vq_body(x_ref, e_ref, e2_ref, q_ref, loss_ref):
    g = pl.program_id(0)
    emb = e_ref[...]
    emb2_bf = e2_ref[...]
    e_sq = jnp.sum(emb * emb, axis=1, keepdims=True)
    partial = jnp.zeros((), jnp.float32)
    for i in range(_BPG):
        q_t, psum = _vq_one_batch(x_ref[i], emb2_bf, e_sq)
        q_ref[i] = q_t
        partial = partial + psum

    @pl.when(g == 0)
    def _init():
        loss_ref[...] = jnp.zeros((1, 1), jnp.float32)

    loss_ref[...] += partial.reshape(1, 1)


@functools.partial(jax.jit, static_argnames=())
def kernel(x_latent, embed_weight):
    B, C, H, W = x_latent.shape
    N = H * W
    x3 = x_latent.reshape(B, C, N)
    emb2_bf = (embed_weight * 2).astype(jnp.bfloat16)
    q3, loss_sum = pl.pallas_call(
        _vq_body,
        grid=(B // _BPG,),
        in_specs=[
            pl.BlockSpec((_BPG, C, N), lambda g: (g, 0, 0)),
            pl.BlockSpec((_NUM_EMBEDS, _EMBED_DIM), lambda g: (0, 0)),
            pl.BlockSpec((_NUM_EMBEDS, _EMBED_DIM), lambda g: (0, 0)),
        ],
        out_specs=[
            pl.BlockSpec((_BPG, C, N), lambda g: (g, 0, 0)),
            pl.BlockSpec((1, 1), lambda g: (0, 0)),
        ],
        out_shape=[
            jax.ShapeDtypeStruct((B, C, N), jnp.float32),
            jax.ShapeDtypeStruct((1, 1), jnp.float32),
        ],
    )(x3, embed_weight, emb2_bf)
    vq_loss = (1.0 + _BETA) * loss_sum[0, 0] / (B * C * H * W)
    return q3.reshape(B, C, H, W), vq_loss
